# baseline (speedup 1.0000x reference)
import jax
import jax.numpy as jnp
import numpy as np
from jax.experimental import pallas as pl
from jax.experimental.pallas import tpu as pltpu

NUM_BUCKETS = 32
NUM_HEADS = 16
Q = 2048
K = 2048
TBL_W = 4096
ROWS = 256
TW = 3840
OFF = 2040
OFFR = 1792


def _build_tbl(embT_ref, h, t8_ref, tbl_ref):
    s = jax.lax.broadcasted_iota(jnp.int32, (8, TBL_W), 0)
    x = jax.lax.broadcasted_iota(jnp.int32, (8, TBL_W), 1)
    d = x - OFF - s
    n = -d
    ret = jnp.where(n < 0, 16, 0)
    n = jnp.abs(n)
    is_small = n < 8
    n_safe = jnp.maximum(n, 1).astype(jnp.float32)
    val = 8 + (jnp.log(n_safe / 8) / np.log(128 / 8) * 8).astype(jnp.int32)
    val = jnp.minimum(val, 15)
    b = ret + jnp.where(is_small, n, val)
    acc = jnp.zeros((8, TBL_W), jnp.float32)
    for j in range(NUM_BUCKETS):
        acc = acc + jnp.where(b == j, embT_ref[h, 0, j], 0.0)
    t8_ref[...] = acc
    for g in range(ROWS // 8):
        base = (OFF - OFFR) - 8 * g
        tbl_ref[8 * g:8 * g + 8, :] = t8_ref[:, base:base + TW]


def _pbe_kernel(embT_ref, out_ref, t8_ref, tbl_ref, sem):
    copies = [[], []]
    for h in range(NUM_HEADS):
        p = h % 2
        for c in copies[p]:
            c.wait()
        copies[p] = []
        _build_tbl(embT_ref, h, t8_ref, tbl_ref.at[p])
        for a in range(Q // ROWS):
            c = pltpu.make_async_copy(
                tbl_ref.at[p, :, pl.ds(OFFR - ROWS * a, K)],
                out_ref.at[h, pl.ds(ROWS * a, ROWS), :],
                sem.at[p],
            )
            c.start()
            copies[p].append(c)
    for p in (0, 1):
        for c in copies[p]:
            c.wait()


def kernel(embedding, query_length, key_length):
    del query_length, key_length
    embT = embedding.T.reshape(NUM_HEADS, 1, NUM_BUCKETS)
    out = pl.pallas_call(
        _pbe_kernel,
        in_specs=[pl.BlockSpec(memory_space=pltpu.VMEM)],
        out_specs=pl.BlockSpec(memory_space=pl.ANY),
        out_shape=jax.ShapeDtypeStruct((NUM_HEADS, Q, K), jnp.float32),
        scratch_shapes=[
            pltpu.VMEM((8, TBL_W), jnp.float32),
            pltpu.VMEM((2, ROWS, TW), jnp.float32),
            pltpu.SemaphoreType.DMA((2,)),
        ],
    )(embT)
    return out[None]


# speedup vs baseline: 1.0107x; 1.0107x over previous
import jax
import jax.numpy as jnp
import numpy as np
from jax.experimental import pallas as pl
from jax.experimental.pallas import tpu as pltpu

NUM_BUCKETS = 32
NUM_HEADS = 16
Q = 2048
K = 2048
TBL_W = 4096
T128_W = 3968
OFF = 2040
OFF128 = 1920


def _build_t128(embT_ref, h, t8_ref, t128_ref):
    s = jax.lax.broadcasted_iota(jnp.int32, (8, TBL_W), 0)
    x = jax.lax.broadcasted_iota(jnp.int32, (8, TBL_W), 1)
    d = x - OFF - s
    n = -d
    ret = jnp.where(n < 0, 16, 0)
    n = jnp.abs(n)
    is_small = n < 8
    n_safe = jnp.maximum(n, 1).astype(jnp.float32)
    val = 8 + (jnp.log(n_safe / 8) / np.log(128 / 8) * 8).astype(jnp.int32)
    val = jnp.minimum(val, 15)
    b = ret + jnp.where(is_small, n, val)
    acc = jnp.zeros((8, TBL_W), jnp.float32)
    for j in range(NUM_BUCKETS):
        acc = acc + jnp.where(b == j, embT_ref[h, 0, j], 0.0)
    t8_ref[...] = acc
    for g in range(16):
        t128_ref[8 * g:8 * g + 8, :] = t8_ref[:, 120 - 8 * g:120 - 8 * g + T128_W]


def _pbe_kernel(embT_ref, out_ref, t8_ref, t128_ref, sem):
    copies = [[], []]
    for h in range(NUM_HEADS):
        p = h % 2
        for c in copies[p]:
            c.wait()
        copies[p] = []
        _build_t128(embT_ref, h, t8_ref, t128_ref.at[p])
        for a in range(Q // 128):
            c = pltpu.make_async_copy(
                t128_ref.at[p, :, pl.ds(OFF128 - 128 * a, K)],
                out_ref.at[h, pl.ds(128 * a, 128), :],
                sem.at[p],
            )
            c.start()
            copies[p].append(c)
    for p in (0, 1):
        for c in copies[p]:
            c.wait()


def kernel(embedding, query_length, key_length):
    del query_length, key_length
    embT = embedding.T.reshape(NUM_HEADS, 1, NUM_BUCKETS)
    out = pl.pallas_call(
        _pbe_kernel,
        in_specs=[pl.BlockSpec(memory_space=pltpu.VMEM)],
        out_specs=pl.BlockSpec(memory_space=pl.ANY),
        out_shape=jax.ShapeDtypeStruct((NUM_HEADS, Q, K), jnp.float32),
        scratch_shapes=[
            pltpu.VMEM((8, TBL_W), jnp.float32),
            pltpu.VMEM((2, 128, T128_W), jnp.float32),
            pltpu.SemaphoreType.DMA((2,)),
        ],
    )(embT)
    return out[None]
